# Initial kernel scaffold; baseline (speedup 1.0000x reference)
import jax
import jax.numpy as jnp
from jax.experimental import pallas as pl


def kernel(x, edge_index, W1, b1, W2, b2):
    raise NotImplementedError("write your pallas kernel here")


# speedup vs baseline: 44.4693x; 44.4693x over previous
import functools

import jax
import jax.numpy as jnp
from jax import lax
from jax.experimental import pallas as pl
from jax.experimental.pallas import tpu as pltpu
from jax.experimental.pallas import tpu_sc as plsc

N = 10000
NPAD = 10240
E = 320000
D_FEAT = 128
H = 16

NW = 32
CH = 128
NCH_SC = 82
NCH = NCH_SC + 2
ROWS_PER_TILE = NPAD // 16

_mesh = plsc.VectorSubcoreMesh(core_axis_name="c", subcore_axis_name="s")
_sc_params = pltpu.CompilerParams(use_tc_tiling_on_sc=False)


@functools.partial(
    pl.kernel,
    out_type=jax.ShapeDtypeStruct((2, NPAD), jnp.float32),
    mesh=_mesh,
    compiler_params=_sc_params,
    scratch_types=[
        pltpu.VMEM((NCH, CH), jnp.int32),
        pltpu.VMEM((CH,), jnp.float32),
        pltpu.VMEM((ROWS_PER_TILE,), jnp.float32),
        pltpu.VMEM_SHARED((NPAD,), jnp.float32),
    ],
)
def _sc_deg(dstw_hbm, deg_out, idx_d, ones_v, zbuf, deg_sh):
    c = lax.axis_index("c")
    s = lax.axis_index("s")
    wid = c * 16 + s
    pltpu.sync_copy(dstw_hbm.at[wid], idx_d)
    for k in range(CH // 16):
        ones_v[pl.ds(16 * k, 16)] = jnp.full((16,), 1.0, jnp.float32)
    for k in range(ROWS_PER_TILE // 16):
        zbuf[pl.ds(16 * k, 16)] = jnp.zeros((16,), jnp.float32)
    pltpu.sync_copy(zbuf, deg_sh.at[pl.ds(s * ROWS_PER_TILE, ROWS_PER_TILE)])
    plsc.subcore_barrier()

    def body(j, carry):
        pltpu.sync_copy(ones_v, deg_sh.at[idx_d.at[j]], add=True)
        return carry

    lax.fori_loop(0, NCH_SC, body, 0)
    plsc.subcore_barrier()
    sl = pl.ds(s * ROWS_PER_TILE, ROWS_PER_TILE)
    pltpu.sync_copy(deg_sh.at[sl], deg_out.at[c, sl])


@functools.partial(
    pl.kernel,
    out_type=jax.ShapeDtypeStruct((2, NPAD, H), jnp.float32),
    mesh=_mesh,
    compiler_params=_sc_params,
    scratch_types=[
        pltpu.VMEM((NCH, CH), jnp.int32),
        pltpu.VMEM((NCH, CH), jnp.int32),
        pltpu.VMEM((CH, H), jnp.float32),
        pltpu.VMEM((CH, H), jnp.float32),
        pltpu.VMEM_SHARED((NPAD, H), jnp.float32),
        pltpu.SemaphoreType.DMA,
        pltpu.SemaphoreType.DMA,
    ],
)
def _sc_agg(tab_hbm, srcw_hbm, dstw_hbm, zeros_hbm, out_hbm,
            idx_s, idx_d, msg0, msg1, acc_sh, semA, semB):
    c = lax.axis_index("c")
    s = lax.axis_index("s")
    wid = c * 16 + s
    sl = pl.ds(s * ROWS_PER_TILE, ROWS_PER_TILE)
    pltpu.sync_copy(srcw_hbm.at[wid], idx_s)
    pltpu.sync_copy(dstw_hbm.at[wid], idx_d)
    pltpu.sync_copy(zeros_hbm.at[sl], acc_sh.at[sl])
    plsc.subcore_barrier()

    pltpu.async_copy(tab_hbm.at[idx_s.at[0]], msg0, semA)
    pltpu.async_copy(tab_hbm.at[idx_s.at[1]], msg1, semB)

    def body(i, carry):
        j0 = 2 * i
        pltpu.make_async_copy(tab_hbm.at[idx_s.at[j0]], msg0, semA).wait()
        pltpu.sync_copy(msg0, acc_sh.at[idx_d.at[j0]], add=True)
        pltpu.async_copy(tab_hbm.at[idx_s.at[j0 + 2]], msg0, semA)
        pltpu.make_async_copy(tab_hbm.at[idx_s.at[j0 + 1]], msg1, semB).wait()
        pltpu.sync_copy(msg1, acc_sh.at[idx_d.at[j0 + 1]], add=True)
        pltpu.async_copy(tab_hbm.at[idx_s.at[j0 + 3]], msg1, semB)
        return carry

    lax.fori_loop(0, NCH_SC // 2, body, 0)
    pltpu.make_async_copy(tab_hbm.at[idx_s.at[0]], msg0, semA).wait()
    pltpu.make_async_copy(tab_hbm.at[idx_s.at[1]], msg1, semB).wait()
    plsc.subcore_barrier()
    pltpu.sync_copy(acc_sh.at[sl], out_hbm.at[c, sl])


def _tc1_body(x_ref, w1_ref, degp_ref, h1s_ref, dis_ref):
    deg = degp_ref[0] + degp_ref[1]
    dis = jnp.where(deg > 0.0, lax.rsqrt(deg), 0.0)
    dis_ref[...] = dis
    h = jnp.dot(x_ref[...], w1_ref[...], preferred_element_type=jnp.float32)
    h1s_ref[0:N, :] = h * dis[0:N]
    h1s_ref[N:NPAD, :] = jnp.zeros((NPAD - N, H), jnp.float32)


def _tc2_body(pp_ref, dis_ref, b1_ref, w2_ref, h2s_ref):
    acc = pp_ref[0] + pp_ref[1]
    r = jnp.maximum(acc * dis_ref[...] + b1_ref[...], 0.0)
    h2 = jnp.dot(r, w2_ref[...], preferred_element_type=jnp.float32)
    h2s_ref[...] = h2 * dis_ref[...]


def _tc3_body(qp_ref, dis_ref, b2_ref, out_ref):
    acc = qp_ref[0, 0:N, :] + qp_ref[1, 0:N, :]
    out_ref[...] = acc * dis_ref[0:N] + b2_ref[...]


_tc1 = pl.pallas_call(
    _tc1_body,
    out_shape=[jax.ShapeDtypeStruct((NPAD, H), jnp.float32),
               jax.ShapeDtypeStruct((NPAD, 1), jnp.float32)],
)
_tc2 = pl.pallas_call(
    _tc2_body,
    out_shape=jax.ShapeDtypeStruct((NPAD, H), jnp.float32),
)
_tc3 = pl.pallas_call(
    _tc3_body,
    out_shape=jax.ShapeDtypeStruct((N, H), jnp.float32),
)


def kernel(x, edge_index, W1, b1, W2, b2):
    src = edge_index[0].astype(jnp.int32)
    dst = edge_index[1].astype(jnp.int32)
    loop = jnp.arange(N, dtype=jnp.int32)

    n_sc = NW * NCH_SC * CH
    pad = N + (jnp.arange(n_sc - (E + N), dtype=jnp.int32) % (NPAD - N))
    extra = (N + (jnp.arange(NW * 2 * CH, dtype=jnp.int32) % (NPAD - N))
             ).reshape(NW, 2, CH)
    srcw = jnp.concatenate(
        [jnp.concatenate([src, loop, pad]).reshape(NW, NCH_SC, CH), extra], axis=1)
    dstw = jnp.concatenate(
        [jnp.concatenate([dst, loop, pad]).reshape(NW, NCH_SC, CH), extra], axis=1)

    zeros2d = jnp.zeros((NPAD, H), jnp.float32)

    degp = _sc_deg(dstw)
    h1s, dis = _tc1(x, W1, degp.reshape(2, NPAD, 1))
    pp = _sc_agg(h1s, srcw, dstw, zeros2d)
    h2s = _tc2(pp, dis, b1.reshape(1, H), W2)
    qp = _sc_agg(h2s, srcw, dstw, zeros2d)
    return _tc3(qp, dis, b2.reshape(1, H))


# speedup vs baseline: 50.3870x; 1.1331x over previous
import functools

import jax
import jax.numpy as jnp
from jax import lax
from jax.experimental import pallas as pl
from jax.experimental.pallas import tpu as pltpu
from jax.experimental.pallas import tpu_sc as plsc

N = 10000
NPAD = 10240
E = 320000
H = 16

NW = 32
CH = 125
NCH = 80
ROWS_PER_TILE = NPAD // 16

_mesh = plsc.VectorSubcoreMesh(core_axis_name="c", subcore_axis_name="s")
_sc_params = pltpu.CompilerParams(use_tc_tiling_on_sc=False)


@functools.partial(
    pl.kernel,
    out_type=jax.ShapeDtypeStruct((2, NPAD), jnp.float32),
    mesh=_mesh,
    compiler_params=_sc_params,
    scratch_types=[
        pltpu.VMEM((NCH, CH), jnp.int32),
        pltpu.VMEM((128,), jnp.float32),
        pltpu.VMEM((ROWS_PER_TILE,), jnp.float32),
        pltpu.VMEM_SHARED((NPAD,), jnp.float32),
        pltpu.SemaphoreType.DMA,
    ],
)
def _sc_deg(dstw_hbm, deg_out, idx_d, ones_v, zbuf, deg_sh, sem):
    c = lax.axis_index("c")
    s = lax.axis_index("s")
    wid = c * 16 + s
    pltpu.sync_copy(dstw_hbm.at[wid], idx_d)
    for k in range(8):
        ones_v[pl.ds(16 * k, 16)] = jnp.full((16,), 1.0, jnp.float32)
    for k in range(ROWS_PER_TILE // 16):
        zbuf[pl.ds(16 * k, 16)] = jnp.zeros((16,), jnp.float32)
    sl = pl.ds(s * ROWS_PER_TILE, ROWS_PER_TILE)
    pltpu.sync_copy(zbuf, deg_sh.at[sl])
    plsc.subcore_barrier()

    ones = ones_v.at[pl.ds(0, CH)]
    DEPTH = 8

    def start(j, carry):
        pltpu.async_copy(ones, deg_sh.at[idx_d.at[j]], sem, add=True)
        return carry

    def wait_one():
        pltpu.make_async_copy(ones, deg_sh.at[idx_d.at[0]], sem).wait()

    lax.fori_loop(0, DEPTH, start, 0)

    def roll(j, carry):
        wait_one()
        return start(j, carry)

    lax.fori_loop(DEPTH, NCH, roll, 0)

    def drain(j, carry):
        wait_one()
        return carry

    lax.fori_loop(0, DEPTH, drain, 0)
    plsc.subcore_barrier()
    pltpu.sync_copy(deg_sh.at[sl], deg_out.at[c, sl])


@functools.partial(
    pl.kernel,
    out_type=jax.ShapeDtypeStruct((2, NPAD, H), jnp.float32),
    mesh=_mesh,
    compiler_params=_sc_params,
    scratch_types=[
        pltpu.VMEM((NCH, CH), jnp.int32),
        pltpu.VMEM((NCH, CH), jnp.int32),
        pltpu.VMEM((CH, H), jnp.float32),
        pltpu.VMEM((CH, H), jnp.float32),
        pltpu.VMEM((CH, H), jnp.float32),
        pltpu.VMEM((CH, H), jnp.float32),
        pltpu.VMEM_SHARED((NPAD, H), jnp.float32),
        pltpu.SemaphoreType.DMA,
        pltpu.SemaphoreType.DMA,
        pltpu.SemaphoreType.DMA,
        pltpu.SemaphoreType.DMA,
        pltpu.SemaphoreType.DMA,
        pltpu.SemaphoreType.DMA,
        pltpu.SemaphoreType.DMA,
        pltpu.SemaphoreType.DMA,
    ],
)
def _sc_agg(tab_hbm, srcw_hbm, dstw_hbm, zeros_hbm, out_hbm,
            idx_s, idx_d, b0, b1, b2, b3, acc_sh,
            g0, g1, g2, g3, s0, s1, s2, s3):
    c = lax.axis_index("c")
    s = lax.axis_index("s")
    wid = c * 16 + s
    sl = pl.ds(s * ROWS_PER_TILE, ROWS_PER_TILE)
    pltpu.sync_copy(srcw_hbm.at[wid], idx_s)
    pltpu.sync_copy(dstw_hbm.at[wid], idx_d)
    pltpu.sync_copy(zeros_hbm.at[sl], acc_sh.at[sl])
    plsc.subcore_barrier()

    bufs = (b0, b1, b2, b3)
    gsems = (g0, g1, g2, g3)
    ssems = (s0, s1, s2, s3)

    def g_start(j, k):
        pltpu.async_copy(tab_hbm.at[idx_s.at[j]], bufs[k], gsems[k])

    def g_wait(k):
        pltpu.make_async_copy(tab_hbm.at[idx_s.at[0]], bufs[k], gsems[k]).wait()

    def s_start(j, k):
        pltpu.async_copy(bufs[k], acc_sh.at[idx_d.at[j]], ssems[k], add=True)

    def s_wait(k):
        pltpu.make_async_copy(bufs[k], acc_sh.at[idx_d.at[0]], ssems[k]).wait()

    g_start(0, 0)
    g_start(1, 1)
    g_wait(0); s_start(0, 0); g_start(2, 2)
    g_wait(1); s_start(1, 1); g_start(3, 3)

    def body(i, carry):
        j = 4 * i + 2
        for cc in range(4):
            k = (2 + cc) % 4
            g_wait(k)
            s_start(j + cc, k)
            s_wait(cc)
            g_start(j + cc + 2, cc)
        return carry

    lax.fori_loop(0, (NCH - 4) // 4, body, 0)
    g_wait(2); s_start(NCH - 2, 2)
    g_wait(3); s_start(NCH - 1, 3)
    s_wait(0); s_wait(1); s_wait(2); s_wait(3)
    plsc.subcore_barrier()
    pltpu.sync_copy(acc_sh.at[sl], out_hbm.at[c, sl])


def _tc1_body(x_ref, w1_ref, degp_ref, h1s_ref, dis_ref):
    deg = degp_ref[0] + degp_ref[1] + 1.0
    dis = lax.rsqrt(deg)
    dis_ref[...] = dis
    h = jnp.dot(x_ref[...], w1_ref[...], preferred_element_type=jnp.float32)
    h1s_ref[0:N, :] = h * dis[0:N]
    h1s_ref[N:NPAD, :] = jnp.zeros((NPAD - N, H), jnp.float32)


def _tc2_body(pp_ref, h1s_ref, dis_ref, b1_ref, w2_ref, h2s_ref):
    acc = pp_ref[0] + pp_ref[1] + h1s_ref[...]
    r = jnp.maximum(acc * dis_ref[...] + b1_ref[...], 0.0)
    h2 = jnp.dot(r, w2_ref[...], preferred_element_type=jnp.float32)
    h2s_ref[0:N, :] = (h2 * dis_ref[...])[0:N]
    h2s_ref[N:NPAD, :] = jnp.zeros((NPAD - N, H), jnp.float32)


def _tc3_body(qp_ref, h2s_ref, dis_ref, b2_ref, out_ref):
    acc = qp_ref[0, 0:N, :] + qp_ref[1, 0:N, :] + h2s_ref[0:N, :]
    out_ref[...] = acc * dis_ref[0:N] + b2_ref[...]


_tc1 = pl.pallas_call(
    _tc1_body,
    out_shape=[jax.ShapeDtypeStruct((NPAD, H), jnp.float32),
               jax.ShapeDtypeStruct((NPAD, 1), jnp.float32)],
)
_tc2 = pl.pallas_call(
    _tc2_body,
    out_shape=jax.ShapeDtypeStruct((NPAD, H), jnp.float32),
)
_tc3 = pl.pallas_call(
    _tc3_body,
    out_shape=jax.ShapeDtypeStruct((N, H), jnp.float32),
)


def kernel(x, edge_index, W1, b1, W2, b2):
    srcw = edge_index[0].astype(jnp.int32).reshape(NW, NCH, CH)
    dstw = edge_index[1].astype(jnp.int32).reshape(NW, NCH, CH)
    zeros2d = jnp.zeros((NPAD, H), jnp.float32)

    degp = _sc_deg(dstw)
    h1s, dis = _tc1(x, W1, degp.reshape(2, NPAD, 1))
    pp = _sc_agg(h1s, srcw, dstw, zeros2d)
    h2s = _tc2(pp, h1s, dis, b1.reshape(1, H), W2)
    qp = _sc_agg(h2s, srcw, dstw, zeros2d)
    return _tc3(qp, h2s, dis, b2.reshape(1, H))


# speedup vs baseline: 56.5433x; 1.1222x over previous
import functools

import jax
import jax.numpy as jnp
from jax import lax
from jax.experimental import pallas as pl
from jax.experimental.pallas import tpu as pltpu
from jax.experimental.pallas import tpu_sc as plsc

N = 10000
NPAD = 10240
E = 320000
H = 16

NW = 32
CH = 125
NCH = 80
ROWS_PER_TILE = NPAD // 16

_mesh = plsc.VectorSubcoreMesh(core_axis_name="c", subcore_axis_name="s")
_sc_params = pltpu.CompilerParams(use_tc_tiling_on_sc=False)


@functools.partial(
    pl.kernel,
    out_type=jax.ShapeDtypeStruct((2, NPAD), jnp.float32),
    mesh=_mesh,
    compiler_params=_sc_params,
    scratch_types=[
        pltpu.VMEM((NCH, CH), jnp.int32),
        pltpu.VMEM((128,), jnp.float32),
        pltpu.VMEM((ROWS_PER_TILE,), jnp.float32),
        pltpu.VMEM_SHARED((NPAD,), jnp.float32),
        pltpu.SemaphoreType.DMA,
    ],
)
def _sc_deg(ei_hbm, deg_out, idx_d, ones_v, zbuf, deg_sh, sem):
    c = lax.axis_index("c")
    s = lax.axis_index("s")
    wid = c * 16 + s
    pltpu.sync_copy(ei_hbm.at[1, wid], idx_d)
    for k in range(8):
        ones_v[pl.ds(16 * k, 16)] = jnp.full((16,), 1.0, jnp.float32)
    for k in range(ROWS_PER_TILE // 16):
        zbuf[pl.ds(16 * k, 16)] = jnp.zeros((16,), jnp.float32)
    sl = pl.ds(s * ROWS_PER_TILE, ROWS_PER_TILE)
    pltpu.sync_copy(zbuf, deg_sh.at[sl])
    plsc.subcore_barrier()

    ones = ones_v.at[pl.ds(0, CH)]
    DEPTH = 8

    def start(j, carry):
        pltpu.async_copy(ones, deg_sh.at[idx_d.at[j]], sem, add=True)
        return carry

    def wait_one():
        pltpu.make_async_copy(ones, deg_sh.at[idx_d.at[0]], sem).wait()

    lax.fori_loop(0, DEPTH, start, 0)

    def roll(j, carry):
        wait_one()
        return start(j, carry)

    lax.fori_loop(DEPTH, NCH, roll, 0)

    def drain(j, carry):
        wait_one()
        return carry

    lax.fori_loop(0, DEPTH, drain, 0)
    plsc.subcore_barrier()
    pltpu.sync_copy(deg_sh.at[sl], deg_out.at[c, sl])


@functools.partial(
    pl.kernel,
    out_type=jax.ShapeDtypeStruct((2, NPAD, H), jnp.float32),
    mesh=_mesh,
    compiler_params=_sc_params,
    scratch_types=[
        pltpu.VMEM((NCH, CH), jnp.int32),
        pltpu.VMEM((NCH, CH), jnp.int32),
        pltpu.VMEM((CH, H), jnp.float32),
        pltpu.VMEM((CH, H), jnp.float32),
        pltpu.VMEM((CH, H), jnp.float32),
        pltpu.VMEM((CH, H), jnp.float32),
        pltpu.VMEM_SHARED((NPAD, H), jnp.float32),
        pltpu.SemaphoreType.DMA,
        pltpu.SemaphoreType.DMA,
        pltpu.SemaphoreType.DMA,
        pltpu.SemaphoreType.DMA,
        pltpu.SemaphoreType.DMA,
        pltpu.SemaphoreType.DMA,
        pltpu.SemaphoreType.DMA,
        pltpu.SemaphoreType.DMA,
    ],
)
def _sc_agg(tab_hbm, ei_hbm, zeros_hbm, out_hbm,
            idx_s, idx_d, b0, b1, b2, b3, acc_sh,
            g0, g1, g2, g3, s0, s1, s2, s3):
    c = lax.axis_index("c")
    s = lax.axis_index("s")
    wid = c * 16 + s
    sl = pl.ds(s * ROWS_PER_TILE, ROWS_PER_TILE)
    pltpu.sync_copy(ei_hbm.at[0, wid], idx_s)
    pltpu.sync_copy(ei_hbm.at[1, wid], idx_d)
    pltpu.sync_copy(zeros_hbm.at[sl], acc_sh.at[sl])
    plsc.subcore_barrier()

    bufs = (b0, b1, b2, b3)
    gsems = (g0, g1, g2, g3)
    ssems = (s0, s1, s2, s3)

    def g_start(j, k):
        pltpu.async_copy(tab_hbm.at[idx_s.at[j]], bufs[k], gsems[k])

    def g_wait(k):
        pltpu.make_async_copy(tab_hbm.at[idx_s.at[0]], bufs[k], gsems[k]).wait()

    def s_start(j, k):
        pltpu.async_copy(bufs[k], acc_sh.at[idx_d.at[j]], ssems[k], add=True)

    def s_wait(k):
        pltpu.make_async_copy(bufs[k], acc_sh.at[idx_d.at[0]], ssems[k]).wait()

    g_start(0, 0)
    g_start(1, 1)
    g_wait(0); s_start(0, 0); g_start(2, 2)
    g_wait(1); s_start(1, 1); g_start(3, 3)

    def body(i, carry):
        j = 4 * i + 2
        for cc in range(4):
            k = (2 + cc) % 4
            g_wait(k)
            s_start(j + cc, k)
            s_wait(cc)
            g_start(j + cc + 2, cc)
        return carry

    lax.fori_loop(0, (NCH - 4) // 4, body, 0)
    g_wait(2); s_start(NCH - 2, 2)
    g_wait(3); s_start(NCH - 1, 3)
    s_wait(0); s_wait(1); s_wait(2); s_wait(3)
    plsc.subcore_barrier()
    pltpu.sync_copy(acc_sh.at[sl], out_hbm.at[c, sl])


def _tc1a_body(x_ref, w1_ref, h1_ref):
    h1_ref[0:N, :] = jnp.dot(x_ref[...], w1_ref[...],
                             preferred_element_type=jnp.float32)
    h1_ref[N:NPAD, :] = jnp.zeros((NPAD - N, H), jnp.float32)


def _tc1b_body(h1_ref, degp_ref, h1s_ref, dis_ref):
    deg = degp_ref[0:1, :] + degp_ref[1:2, :] + 1.0
    dis_col = jnp.transpose(lax.rsqrt(deg), (1, 0))
    dis16 = jnp.broadcast_to(dis_col, (NPAD, H))
    dis_ref[...] = dis16
    h1s_ref[...] = h1_ref[...] * dis16


def _tc2_body(pp_ref, h1s_ref, dis_ref, b1_ref, w2_ref, h2s_ref):
    acc = pp_ref[0] + pp_ref[1] + h1s_ref[...]
    r = jnp.maximum(acc * dis_ref[...] + b1_ref[...], 0.0)
    h2 = jnp.dot(r, w2_ref[...], preferred_element_type=jnp.float32)
    h2s_ref[0:N, :] = (h2 * dis_ref[...])[0:N]
    h2s_ref[N:NPAD, :] = jnp.zeros((NPAD - N, H), jnp.float32)


def _tc3_body(qp_ref, h2s_ref, dis_ref, b2_ref, out_ref):
    acc = qp_ref[0, 0:N, :] + qp_ref[1, 0:N, :] + h2s_ref[0:N, :]
    out_ref[...] = acc * dis_ref[0:N, :] + b2_ref[...]


_tc1a = pl.pallas_call(
    _tc1a_body,
    out_shape=jax.ShapeDtypeStruct((NPAD, H), jnp.float32),
)
_tc1b = pl.pallas_call(
    _tc1b_body,
    out_shape=[jax.ShapeDtypeStruct((NPAD, H), jnp.float32),
               jax.ShapeDtypeStruct((NPAD, H), jnp.float32)],
)
_tc2 = pl.pallas_call(
    _tc2_body,
    out_shape=jax.ShapeDtypeStruct((NPAD, H), jnp.float32),
)
_tc3 = pl.pallas_call(
    _tc3_body,
    out_shape=jax.ShapeDtypeStruct((N, H), jnp.float32),
)


def kernel(x, edge_index, W1, b1, W2, b2):
    ei = edge_index.astype(jnp.int32).reshape(2, NW, NCH, CH)
    zeros2d = jnp.zeros((NPAD, H), jnp.float32)

    degp = _sc_deg(ei)
    h1 = _tc1a(x, W1)
    h1s, dis = _tc1b(h1, degp)
    pp = _sc_agg(h1s, ei, zeros2d)
    h2s = _tc2(pp, h1s, dis, b1.reshape(1, H), W2)
    qp = _sc_agg(h2s, ei, zeros2d)
    return _tc3(qp, h2s, dis, b2.reshape(1, H))


# speedup vs baseline: 74.0237x; 1.3092x over previous
import functools

import jax
import jax.numpy as jnp
from jax import lax
from jax.experimental import pallas as pl
from jax.experimental.pallas import tpu as pltpu
from jax.experimental.pallas import tpu_sc as plsc

N = 10000
NPAD = 10240
E = 320000
H = 16

NW = 32
CH = 125
NCH = 80
ROWS_PER_TILE = NPAD // 16

_mesh = plsc.VectorSubcoreMesh(core_axis_name="c", subcore_axis_name="s")
_sc_params = pltpu.CompilerParams(use_tc_tiling_on_sc=False)


@functools.partial(
    pl.kernel,
    out_type=jax.ShapeDtypeStruct((2, NPAD, H), jnp.float32),
    mesh=_mesh,
    compiler_params=_sc_params,
    scratch_types=[
        pltpu.VMEM((NCH, CH), jnp.int32),
        pltpu.VMEM((128,), jnp.float32),
        pltpu.VMEM((ROWS_PER_TILE,), jnp.float32),
        pltpu.VMEM((ROWS_PER_TILE, H), jnp.float32),
        pltpu.VMEM_SHARED((NPAD,), jnp.float32),
        pltpu.SemaphoreType.DMA,
    ],
)
def _sc_deg(ei_hbm, deg_out, idx_d, ones_v, zbuf, dv16, deg_sh, sem):
    c = lax.axis_index("c")
    s = lax.axis_index("s")
    wid = c * 16 + s
    pltpu.sync_copy(ei_hbm.at[1, wid], idx_d)
    for k in range(8):
        ones_v[pl.ds(16 * k, 16)] = jnp.full((16,), 1.0, jnp.float32)
    for k in range(ROWS_PER_TILE // 16):
        zbuf[pl.ds(16 * k, 16)] = jnp.zeros((16,), jnp.float32)
    sl = pl.ds(s * ROWS_PER_TILE, ROWS_PER_TILE)
    pltpu.sync_copy(zbuf, deg_sh.at[sl])
    plsc.subcore_barrier()

    ones = ones_v.at[pl.ds(0, CH)]
    DEPTH = 8

    def start(j, carry):
        pltpu.async_copy(ones, deg_sh.at[idx_d.at[j]], sem, add=True)
        return carry

    def wait_one():
        pltpu.make_async_copy(ones, deg_sh.at[idx_d.at[0]], sem).wait()

    lax.fori_loop(0, DEPTH, start, 0)

    def roll(j, carry):
        wait_one()
        return start(j, carry)

    lax.fori_loop(DEPTH, NCH, roll, 0)

    def drain(j, carry):
        wait_one()
        return carry

    lax.fori_loop(0, DEPTH, drain, 0)
    plsc.subcore_barrier()
    pltpu.sync_copy(deg_sh.at[sl], zbuf)

    def bc(g, carry):
        dvec = zbuf[pl.ds(16 * g, 16)]
        for j in range(16):
            dv16[16 * g + j, :] = jnp.broadcast_to(dvec[j], (H,))
        return carry

    lax.fori_loop(0, ROWS_PER_TILE // 16, bc, 0)
    pltpu.sync_copy(dv16, deg_out.at[c, sl])


@functools.partial(
    pl.kernel,
    out_type=jax.ShapeDtypeStruct((2, NPAD, H), jnp.float32),
    mesh=_mesh,
    compiler_params=_sc_params,
    scratch_types=[
        pltpu.VMEM((NCH, CH), jnp.int32),
        pltpu.VMEM((NCH, CH), jnp.int32),
        pltpu.VMEM((CH, H), jnp.float32),
        pltpu.VMEM((CH, H), jnp.float32),
        pltpu.VMEM((CH, H), jnp.float32),
        pltpu.VMEM((CH, H), jnp.float32),
        pltpu.VMEM_SHARED((NPAD, H), jnp.float32),
        pltpu.SemaphoreType.DMA,
        pltpu.SemaphoreType.DMA,
        pltpu.SemaphoreType.DMA,
        pltpu.SemaphoreType.DMA,
        pltpu.SemaphoreType.DMA,
        pltpu.SemaphoreType.DMA,
        pltpu.SemaphoreType.DMA,
        pltpu.SemaphoreType.DMA,
    ],
)
def _sc_agg(tab_hbm, ei_hbm, zeros_hbm, out_hbm,
            idx_s, idx_d, b0, b1, b2, b3, acc_sh,
            g0, g1, g2, g3, s0, s1, s2, s3):
    c = lax.axis_index("c")
    s = lax.axis_index("s")
    wid = c * 16 + s
    sl = pl.ds(s * ROWS_PER_TILE, ROWS_PER_TILE)
    pltpu.sync_copy(ei_hbm.at[0, wid], idx_s)
    pltpu.sync_copy(ei_hbm.at[1, wid], idx_d)
    pltpu.sync_copy(zeros_hbm.at[sl], acc_sh.at[sl])
    plsc.subcore_barrier()

    bufs = (b0, b1, b2, b3)
    gsems = (g0, g1, g2, g3)
    ssems = (s0, s1, s2, s3)

    def g_start(j, k):
        pltpu.async_copy(tab_hbm.at[idx_s.at[j]], bufs[k], gsems[k])

    def g_wait(k):
        pltpu.make_async_copy(tab_hbm.at[idx_s.at[0]], bufs[k], gsems[k]).wait()

    def s_start(j, k):
        pltpu.async_copy(bufs[k], acc_sh.at[idx_d.at[j]], ssems[k], add=True)

    def s_wait(k):
        pltpu.make_async_copy(bufs[k], acc_sh.at[idx_d.at[0]], ssems[k]).wait()

    g_start(0, 0)
    g_start(1, 1)
    g_wait(0); s_start(0, 0); g_start(2, 2)
    g_wait(1); s_start(1, 1); g_start(3, 3)

    def body(i, carry):
        j = 4 * i + 2
        for cc in range(4):
            k = (2 + cc) % 4
            g_wait(k)
            s_start(j + cc, k)
            s_wait(cc)
            g_start(j + cc + 2, cc)
        return carry

    lax.fori_loop(0, (NCH - 4) // 4, body, 0)
    g_wait(2); s_start(NCH - 2, 2)
    g_wait(3); s_start(NCH - 1, 3)
    s_wait(0); s_wait(1); s_wait(2); s_wait(3)
    plsc.subcore_barrier()
    pltpu.sync_copy(acc_sh.at[sl], out_hbm.at[c, sl])


NR = NPAD // 8
NRV = N // 8

def _tc1a_body(x8_ref, w1b_ref, h1_ref):
    h1_ref[0:NRV, :] = jnp.dot(x8_ref[...], w1b_ref[...],
                               preferred_element_type=jnp.float32)
    h1_ref[NRV:NR, :] = jnp.zeros((NR - NRV, 128), jnp.float32)


def _tc1b_body(h1_ref, degp_ref, h1s_ref, dis_ref):
    dis = lax.rsqrt(degp_ref[0] + degp_ref[1] + 1.0)
    dis_ref[...] = dis
    h1s_ref[...] = h1_ref[...] * dis


def _tc2_body(pp_ref, h1s_ref, dis_ref, b1_ref, w2b_ref, h2s_ref):
    acc = pp_ref[0] + pp_ref[1] + h1s_ref[...]
    r = jnp.maximum(acc * dis_ref[...] + b1_ref[...], 0.0)
    h2 = jnp.dot(r, w2b_ref[...], preferred_element_type=jnp.float32)
    h2s_ref[0:NRV, :] = (h2 * dis_ref[...])[0:NRV]
    h2s_ref[NRV:NR, :] = jnp.zeros((NR - NRV, 128), jnp.float32)


def _tc3_body(qp_ref, h2s_ref, dis_ref, b2_ref, out_ref):
    acc = qp_ref[0, 0:NRV, :] + qp_ref[1, 0:NRV, :] + h2s_ref[0:NRV, :]
    out_ref[...] = acc * dis_ref[0:NRV, :] + b2_ref[...]


_tc1a = pl.pallas_call(
    _tc1a_body,
    out_shape=jax.ShapeDtypeStruct((NR, 128), jnp.float32),
)
_tc1b = pl.pallas_call(
    _tc1b_body,
    out_shape=[jax.ShapeDtypeStruct((NR, 128), jnp.float32),
               jax.ShapeDtypeStruct((NR, 128), jnp.float32)],
)
_tc2 = pl.pallas_call(
    _tc2_body,
    out_shape=jax.ShapeDtypeStruct((NR, 128), jnp.float32),
)
_tc3 = pl.pallas_call(
    _tc3_body,
    out_shape=jax.ShapeDtypeStruct((NRV, 128), jnp.float32),
)


def kernel(x, edge_index, W1, b1, W2, b2):
    f32 = jnp.float32
    ei = edge_index.astype(jnp.int32).reshape(2, NW, NCH, CH)
    zeros2d = jnp.zeros((NPAD, H), f32)
    eye8 = jnp.eye(8, dtype=f32)
    w1b = jnp.kron(eye8, W1.astype(f32))
    w2b = jnp.kron(eye8, W2.astype(f32))
    b1w = jnp.tile(b1.astype(f32), 8).reshape(1, 128)
    b2w = jnp.tile(b2.astype(f32), 8).reshape(1, 128)
    x8 = x.astype(f32).reshape(NRV, 8 * 128)

    degp = _sc_deg(ei)
    h1 = _tc1a(x8, w1b)
    h1s, dis = _tc1b(h1, degp.reshape(2, NR, 128))
    pp = _sc_agg(h1s.reshape(NPAD, H), ei, zeros2d)
    h2s = _tc2(pp.reshape(2, NR, 128), h1s, dis, b1w, w2b)
    qp = _sc_agg(h2s.reshape(NPAD, H), ei, zeros2d)
    out = _tc3(qp.reshape(2, NR, 128), h2s, dis, b2w)
    return out.reshape(N, H)
